# baseline (speedup 1.0000x reference)
import functools

import jax
import jax.numpy as jnp
from jax import lax
from jax.experimental import pallas as pl
from jax.experimental.pallas import tpu as pltpu
from jax.experimental.pallas import tpu_sc as plsc

NUM_CORES = 2
NUM_SUBCORES = 16
NUM_TILES = NUM_CORES * NUM_SUBCORES
CHUNK = 80
ROW_BLK = 1000


def _spmm_sc(feat, src, dst, w):
    n, d = feat.shape
    e = src.shape[0]
    epw = e // NUM_TILES
    n_chunks = epw // CHUNK
    n_pieces = 5
    S = n_chunks // n_pieces
    stripe_stride = (n // NUM_SUBCORES) // 8 * 8
    stripe_width = n - (NUM_SUBCORES - 1) * stripe_stride
    mesh = plsc.VectorSubcoreMesh(
        core_axis_name="c", subcore_axis_name="s",
        num_cores=NUM_CORES, num_subcores=NUM_SUBCORES)

    @functools.partial(
        pl.kernel,
        out_type=jax.ShapeDtypeStruct((NUM_CORES, n, d), jnp.float32),
        mesh=mesh,
        scratch_types=[
            [pltpu.VMEM((S, CHUNK), jnp.int32)] * 2,
            [pltpu.VMEM((S, CHUNK), jnp.int32)] * 2,
            [pltpu.VMEM((S, CHUNK), jnp.float32)] * 2,
            pltpu.VMEM((CHUNK, d), jnp.float32),
            pltpu.VMEM((CHUNK, d), jnp.float32),
            pltpu.VMEM_SHARED((n, d), jnp.float32),
            [pltpu.SemaphoreType.DMA] * 2,
            pltpu.SemaphoreType.DMA,
            pltpu.SemaphoreType.DMA,
        ],
        compiler_params=pltpu.CompilerParams(use_tc_tiling_on_sc=False),
    )
    def k(feat_hbm, src_hbm, dst_hbm, w_hbm, zeros_hbm, out_hbm,
          sbuf, dbuf, wbuf, rows_g, rows_s, acc_sh, sem_p, sem_g, sem_s):
        c = lax.axis_index("c")
        s = lax.axis_index("s")
        wid = s * NUM_CORES + c

        def start_piece(pi, b):
            pltpu.async_copy(src_hbm.at[wid, pi], sbuf[b], sem_p[b])
            pltpu.async_copy(dst_hbm.at[wid, pi], dbuf[b], sem_p[b])
            pltpu.async_copy(w_hbm.at[wid, pi], wbuf[b], sem_p[b])

        def wait_piece(b):
            pltpu.make_async_copy(src_hbm.at[wid, 0], sbuf[b], sem_p[b]).wait()
            pltpu.make_async_copy(dst_hbm.at[wid, 0], dbuf[b], sem_p[b]).wait()
            pltpu.make_async_copy(w_hbm.at[wid, 0], wbuf[b], sem_p[b]).wait()

        def start_gather(b, cj):
            pltpu.async_copy(feat_hbm.at[sbuf[b].at[cj]], rows_g, sem_g)

        def wait_gather():
            pltpu.make_async_copy(feat_hbm.at[sbuf[0].at[0]], rows_g,
                                  sem_g).wait()

        def scale(b, cj):
            @pl.loop(0, CHUNK)
            def _(j):
                jbase = (j // 16) * 16
                w16 = wbuf[b][cj, pl.ds(jbase, 16)]
                wvec = w16[jnp.full((16,), j, jnp.int32) - jbase]
                for kk in range(d // 16):
                    sl = (j, pl.ds(kk * 16, 16))
                    rows_s[sl] = rows_g[sl] * wvec

        def start_scatter(b, cj):
            pltpu.async_copy(rows_s, acc_sh.at[dbuf[b].at[cj]], sem_s,
                             add=True)

        def wait_scatter():
            pltpu.make_async_copy(rows_s, acc_sh.at[dbuf[0].at[0]],
                                  sem_s).wait()

        start_piece(0, 0)
        start_piece(1, 1)
        r0 = s * stripe_stride
        pltpu.sync_copy(zeros_hbm.at[pl.ds(r0, stripe_width)],
                        acc_sh.at[pl.ds(r0, stripe_width)])
        plsc.subcore_barrier()
        wait_piece(0)
        start_gather(0, 0)

        for pi in range(n_pieces):
            pb, nb = pi % 2, (pi + 1) % 2
            wait_gather()
            scale(pb, 0)
            start_gather(pb, 1)
            start_scatter(pb, 0)

            @pl.loop(1, S - 1)
            def _(cj):
                wait_gather()
                scale(pb, cj)
                start_gather(pb, cj + 1)
                wait_scatter()
                start_scatter(pb, cj)

            wait_gather()
            scale(pb, S - 1)
            if pi + 1 < n_pieces:
                wait_piece(nb)
                start_gather(nb, 0)
            wait_scatter()
            start_scatter(pb, S - 1)
            wait_scatter()
            if pi + 2 < n_pieces:
                start_piece(pi + 2, pb)

        plsc.subcore_barrier()
        pltpu.sync_copy(acc_sh.at[pl.ds(r0, stripe_width)],
                        out_hbm.at[c, pl.ds(r0, stripe_width)])

    e4 = (NUM_TILES, n_pieces, S, CHUNK)
    return k(feat, src.reshape(e4), dst.reshape(e4), w.reshape(e4),
             jnp.zeros((n, d), jnp.float32))


def _matmul_tc(x, w):
    n, kdim = x.shape
    m = w.shape[1]

    def body(x_ref, w_ref, o_ref):
        o_ref[...] = jnp.dot(x_ref[...], w_ref[...],
                             preferred_element_type=jnp.float32)

    return pl.pallas_call(
        body,
        grid=(n // ROW_BLK,),
        in_specs=[
            pl.BlockSpec((ROW_BLK, kdim), lambda i: (i, 0)),
            pl.BlockSpec((kdim, m), lambda i: (0, 0)),
        ],
        out_specs=pl.BlockSpec((ROW_BLK, m), lambda i: (i, 0)),
        out_shape=jax.ShapeDtypeStruct((n, m), jnp.float32),
    )(x, w)


def _combine_relu_matmul_tc(p, b, w):
    _, n, kdim = p.shape
    m = w.shape[1]

    def body(p_ref, b_ref, w_ref, o_ref):
        h = jnp.maximum(p_ref[0] + p_ref[1] + b_ref[...], 0.0)
        o_ref[...] = jnp.dot(h, w_ref[...],
                             preferred_element_type=jnp.float32)

    return pl.pallas_call(
        body,
        grid=(n // ROW_BLK,),
        in_specs=[
            pl.BlockSpec((2, ROW_BLK, kdim), lambda i: (0, i, 0)),
            pl.BlockSpec((1, kdim), lambda i: (0, 0)),
            pl.BlockSpec((kdim, m), lambda i: (0, 0)),
        ],
        out_specs=pl.BlockSpec((ROW_BLK, m), lambda i: (i, 0)),
        out_shape=jax.ShapeDtypeStruct((n, m), jnp.float32),
    )(p, b.reshape(1, kdim), w)


def _combine_logsoftmax_tc(p, b):
    _, n, m = p.shape

    def body(p_ref, b_ref, o_ref):
        y = p_ref[0] + p_ref[1] + b_ref[...]
        z = y - jnp.max(y, axis=1, keepdims=True)
        o_ref[...] = z - jnp.log(jnp.sum(jnp.exp(z), axis=1, keepdims=True))

    return pl.pallas_call(
        body,
        grid=(n // ROW_BLK,),
        in_specs=[
            pl.BlockSpec((2, ROW_BLK, m), lambda i: (0, i, 0)),
            pl.BlockSpec((1, m), lambda i: (0, 0)),
        ],
        out_specs=pl.BlockSpec((ROW_BLK, m), lambda i: (i, 0)),
        out_shape=jax.ShapeDtypeStruct((n, m), jnp.float32),
    )(p, b.reshape(1, m))


def kernel(x, edge_index, edge_weight, W1, b1, W2, b2):
    src = edge_index[0]
    dst = edge_index[1]
    xw1 = _matmul_tc(x, W1)
    p1 = _spmm_sc(xw1, src, dst, edge_weight)
    hw2 = _combine_relu_matmul_tc(p1, b1, W2)
    p2 = _spmm_sc(hw2, src, dst, edge_weight)
    return _combine_logsoftmax_tc(p2, b2)


# speedup vs baseline: 1.3288x; 1.3288x over previous
import functools

import jax
import jax.numpy as jnp
from jax import lax
from jax.experimental import pallas as pl
from jax.experimental.pallas import tpu as pltpu
from jax.experimental.pallas import tpu_sc as plsc

NUM_CORES = 2
NUM_SUBCORES = 16
NUM_TILES = NUM_CORES * NUM_SUBCORES
CHUNK = 40
ROW_BLK = 1000


def _spmm_sc(feat, src, dst, w):
    n, d = feat.shape
    e = src.shape[0]
    epw = e // NUM_TILES
    n_chunks = epw // CHUNK
    stripe_stride = (n // NUM_SUBCORES) // 8 * 8
    stripe_width = n - (NUM_SUBCORES - 1) * stripe_stride
    mesh = plsc.VectorSubcoreMesh(
        core_axis_name="c", subcore_axis_name="s",
        num_cores=NUM_CORES, num_subcores=NUM_SUBCORES)

    @functools.partial(
        pl.kernel,
        out_type=jax.ShapeDtypeStruct((NUM_CORES, n, d), jnp.float32),
        mesh=mesh,
        scratch_types=[
            pltpu.VMEM((n_chunks, CHUNK), jnp.int32),
            pltpu.VMEM((n_chunks, CHUNK), jnp.int32),
            pltpu.VMEM((n_chunks, CHUNK), jnp.float32),
            [pltpu.VMEM((CHUNK, d), jnp.float32)] * 2,
            [pltpu.VMEM((CHUNK, d), jnp.float32)] * 2,
            pltpu.VMEM_SHARED((n, d), jnp.float32),
            pltpu.SemaphoreType.DMA,
            [pltpu.SemaphoreType.DMA] * 2,
            [pltpu.SemaphoreType.DMA] * 2,
        ],
        compiler_params=pltpu.CompilerParams(use_tc_tiling_on_sc=False),
    )
    def k(feat_hbm, src_hbm, dst_hbm, w_hbm, zeros_hbm, out_hbm,
          src_v, dst_v, w_v, rows_g, rows_s, acc_sh, sem_l, sem_g, sem_s):
        c = lax.axis_index("c")
        s = lax.axis_index("s")
        wid = s * NUM_CORES + c

        def start_gather(ci, b):
            pltpu.async_copy(feat_hbm.at[src_v.at[ci]], rows_g[b], sem_g[b])

        def wait_gather(b):
            pltpu.make_async_copy(feat_hbm.at[src_v.at[0]], rows_g[b],
                                  sem_g[b]).wait()

        def scale(ci, b):
            @pl.loop(0, CHUNK)
            def _(j):
                jbase = jnp.minimum((j // 16) * 16, CHUNK - 16)
                w16 = w_v[ci, pl.ds(jbase, 16)]
                wvec = w16[jnp.full((16,), j, jnp.int32) - jbase]
                for kk in range(d // 16):
                    sl = (j, pl.ds(kk * 16, 16))
                    rows_s[b][sl] = rows_g[b][sl] * wvec

        def start_scatter(ci, b):
            pltpu.async_copy(rows_s[b], acc_sh.at[dst_v.at[ci]], sem_s[b],
                             add=True)

        def wait_scatter(b):
            pltpu.make_async_copy(rows_s[b], acc_sh.at[dst_v.at[0]],
                                  sem_s[b]).wait()

        cp_s = pltpu.async_copy(src_hbm.at[wid], src_v, sem_l)
        cp_d = pltpu.async_copy(dst_hbm.at[wid], dst_v, sem_l)
        cp_w = pltpu.async_copy(w_hbm.at[wid], w_v, sem_l)
        r0 = s * stripe_stride
        pltpu.sync_copy(zeros_hbm.at[pl.ds(r0, stripe_width)],
                        acc_sh.at[pl.ds(r0, stripe_width)])
        cp_s.wait()
        cp_d.wait()
        cp_w.wait()
        plsc.subcore_barrier()
        start_gather(0, 0)
        start_gather(1, 1)

        for i in (0, 1):
            wait_gather(i)
            scale(i, i)
            start_gather(i + 2, i)
            start_scatter(i, i)

        @pl.loop(2, n_chunks - 2, step=2)
        def _(ci):
            for b in range(2):
                i = ci + b
                wait_gather(b)
                wait_scatter(b)
                scale(i, b)
                start_gather(i + 2, b)
                start_scatter(i, b)

        for i in (n_chunks - 2, n_chunks - 1):
            b = i % 2
            wait_gather(b)
            wait_scatter(b)
            scale(i, b)
            start_scatter(i, b)

        wait_scatter(0)
        wait_scatter(1)
        plsc.subcore_barrier()
        pltpu.sync_copy(acc_sh.at[pl.ds(r0, stripe_width)],
                        out_hbm.at[c, pl.ds(r0, stripe_width)])

    e3 = (NUM_TILES, n_chunks, CHUNK)
    return k(feat, src.reshape(e3), dst.reshape(e3), w.reshape(e3),
             jnp.zeros((n, d), jnp.float32))


def _matmul_tc(x, w):
    n, kdim = x.shape
    m = w.shape[1]

    def body(x_ref, w_ref, o_ref):
        o_ref[...] = jnp.dot(x_ref[...], w_ref[...],
                             preferred_element_type=jnp.float32)

    return pl.pallas_call(
        body,
        grid=(n // ROW_BLK,),
        in_specs=[
            pl.BlockSpec((ROW_BLK, kdim), lambda i: (i, 0)),
            pl.BlockSpec((kdim, m), lambda i: (0, 0)),
        ],
        out_specs=pl.BlockSpec((ROW_BLK, m), lambda i: (i, 0)),
        out_shape=jax.ShapeDtypeStruct((n, m), jnp.float32),
    )(x, w)


def _combine_relu_matmul_tc(p, b, w):
    _, n, kdim = p.shape
    m = w.shape[1]

    def body(p_ref, b_ref, w_ref, o_ref):
        h = jnp.maximum(p_ref[0] + p_ref[1] + b_ref[...], 0.0)
        o_ref[...] = jnp.dot(h, w_ref[...],
                             preferred_element_type=jnp.float32)

    return pl.pallas_call(
        body,
        grid=(n // ROW_BLK,),
        in_specs=[
            pl.BlockSpec((2, ROW_BLK, kdim), lambda i: (0, i, 0)),
            pl.BlockSpec((1, kdim), lambda i: (0, 0)),
            pl.BlockSpec((kdim, m), lambda i: (0, 0)),
        ],
        out_specs=pl.BlockSpec((ROW_BLK, m), lambda i: (i, 0)),
        out_shape=jax.ShapeDtypeStruct((n, m), jnp.float32),
    )(p, b.reshape(1, kdim), w)


def _combine_logsoftmax_tc(p, b):
    _, n, m = p.shape

    def body(p_ref, b_ref, o_ref):
        y = p_ref[0] + p_ref[1] + b_ref[...]
        z = y - jnp.max(y, axis=1, keepdims=True)
        o_ref[...] = z - jnp.log(jnp.sum(jnp.exp(z), axis=1, keepdims=True))

    return pl.pallas_call(
        body,
        grid=(n // ROW_BLK,),
        in_specs=[
            pl.BlockSpec((2, ROW_BLK, m), lambda i: (0, i, 0)),
            pl.BlockSpec((1, m), lambda i: (0, 0)),
        ],
        out_specs=pl.BlockSpec((ROW_BLK, m), lambda i: (i, 0)),
        out_shape=jax.ShapeDtypeStruct((n, m), jnp.float32),
    )(p, b.reshape(1, m))


def kernel(x, edge_index, edge_weight, W1, b1, W2, b2):
    src = edge_index[0]
    dst = edge_index[1]
    xw1 = _matmul_tc(x, W1)
    p1 = _spmm_sc(xw1, src, dst, edge_weight)
    hw2 = _combine_relu_matmul_tc(p1, b1, W2)
    p2 = _spmm_sc(hw2, src, dst, edge_weight)
    return _combine_logsoftmax_tc(p2, b2)
